# baseline (speedup 1.0000x reference)
import functools

import jax
import jax.numpy as jnp
from jax import lax
from jax.experimental import pallas as pl
from jax.experimental.pallas import tpu as pltpu
from jax.experimental.pallas import tpu_sc as plsc

_B = 4096
_L = 50
_H = 128
_SYM = 32
_C = 16

_NC = 2
_NS = 16
_NW = _NC * _NS
_BPW = _B // _NW
_PAIRS = _BPW // 2
_PL = 2 * _L
_LANES = 16
_NBUF = 2
_P = 4


def _sum_pool_sc(ids2, didx2, emb_table):
    mesh = plsc.VectorSubcoreMesh(core_axis_name="c", subcore_axis_name="s")

    @functools.partial(
        pl.kernel,
        out_type=jax.ShapeDtypeStruct((_B * _P, _H), jnp.float32),
        mesh=mesh,
        scratch_types=[
            pltpu.VMEM((_PAIRS, _PL), jnp.int32),
            pltpu.VMEM((_PAIRS, _PL), jnp.int32),
            pltpu.VMEM((_NBUF, _PL, _H), jnp.float32),
            pltpu.VMEM((_BPW, _H), jnp.float32),
            pltpu.VMEM_SHARED((_NS * _BPW * _P, _H), jnp.float32),
            [pltpu.SemaphoreType.DMA] * _NBUF,
            [pltpu.SemaphoreType.DMA] * _NBUF,
        ],
    )
    def body(ids_hbm, didx_hbm, table_hbm, out_hbm,
             ids_v, didx_v, rows_v, z_v, acc_sh, sems, ssems):
        c = lax.axis_index("c")
        s = lax.axis_index("s")
        wid = s * _NC + c
        pltpu.sync_copy(ids_hbm.at[pl.ds(wid * _PAIRS, _PAIRS)], ids_v)
        pltpu.sync_copy(didx_hbm.at[pl.ds(wid * _PAIRS, _PAIRS)], didx_v)

        def zbody(r, carry):
            for k in range(_H // _LANES):
                z_v[r, pl.ds(k * _LANES, _LANES)] = jnp.zeros(
                    (_LANES,), jnp.float32
                )
            return carry

        lax.fori_loop(0, _BPW, zbody, 0)
        for t in range(_P):
            pltpu.sync_copy(
                z_v, acc_sh.at[pl.ds(s * _BPW * _P + t * _BPW, _BPW)]
            )

        def issue(pair, buf):
            return pltpu.async_copy(
                table_hbm.at[ids_v.at[pair]], rows_v.at[buf], sems[buf]
            )

        def drain(pair, buf):
            pltpu.make_async_copy(
                table_hbm.at[ids_v.at[pair]], rows_v.at[buf], sems[buf]
            ).wait()

        def scat(pair, buf):
            return pltpu.async_copy(
                rows_v.at[buf], acc_sh.at[didx_v.at[pair]], ssems[buf], add=True
            )

        def scat_drain(pair, buf):
            pltpu.make_async_copy(
                rows_v.at[buf], acc_sh.at[didx_v.at[pair]], ssems[buf]
            ).wait()

        for b in range(_NBUF - 1):
            issue(b, b)

        def step(q, b, first):
            nb = (b + _NBUF - 1) % _NBUF
            if first:
                pl.when(q > 0)(lambda: scat_drain(q - 1, nb))
            else:
                scat_drain(q - 1, nb)
            issue(jnp.minimum(q + _NBUF - 1, _PAIRS - 1), nb)
            drain(q, b)
            scat(q, b)

        def loop_body(i, carry):
            for b in range(_NBUF):
                step(i * _NBUF + b, b, first=(b == 0))
            return carry

        lax.fori_loop(0, _PAIRS // _NBUF, loop_body, 0)
        scat_drain(_PAIRS - 1, (_PAIRS - 1) % _NBUF)
        for b in range(_NBUF - 1):
            drain(_PAIRS - 1, b)

        pltpu.sync_copy(
            acc_sh.at[pl.ds(s * _BPW * _P, _BPW * _P)],
            out_hbm.at[pl.ds(wid * _BPW * _P, _BPW * _P)],
        )

    return body(ids2, didx2, emb_table)


def _fc_body(emb_sum_ref, ids_ref, sym_ref, w1_ref, w2_ref, b_ref, out_ref):
    cnt = jnp.sum(jnp.where(ids_ref[...] != 0, 1.0, 0.0), axis=1, keepdims=True)
    avg = jnp.sum(emb_sum_ref[...], axis=1) * (1.0 / jnp.maximum(cnt, 1.0))
    out_ref[...] = (
        jnp.dot(avg, w1_ref[...], preferred_element_type=jnp.float32)
        + jnp.dot(sym_ref[...], w2_ref[...], preferred_element_type=jnp.float32)
        + b_ref[...]
    )


def kernel(ids, sym, emb_table, fc_w, fc_b):
    ids = ids.astype(jnp.int32)
    ids2 = ids.reshape(_B // 2, _PL)

    p = jnp.arange(_B // 2, dtype=jnp.int32)
    base = (p // _PAIRS // _NC) * _BPW + 2 * (p % _PAIRS)
    i = jnp.arange(_PL, dtype=jnp.int32)
    hi = (i >= _L).astype(jnp.int32)
    part = (i - _L * hi) % _P
    didx2 = (base[:, None] + hi[None, :]) * _P + part[None, :]

    emb_sum = _sum_pool_sc(ids2, didx2, emb_table).reshape(_B, _P, _H)

    w1 = fc_w[:, :_H].T
    w2 = fc_w[:, _H:].T
    out = pl.pallas_call(
        _fc_body,
        out_shape=jax.ShapeDtypeStruct((_B, _C), jnp.float32),
    )(emb_sum, ids, sym, w1, w2, fc_b.reshape(1, _C))
    return out


# speedup vs baseline: 1.0730x; 1.0730x over previous
import functools

import jax
import jax.numpy as jnp
from jax import lax
from jax.experimental import pallas as pl
from jax.experimental.pallas import tpu as pltpu
from jax.experimental.pallas import tpu_sc as plsc

_B = 4096
_L = 50
_H = 128
_SYM = 32
_C = 16

_NC = 2
_NS = 16
_NW = _NC * _NS
_BPW = _B // _NW
_PAIRS = _BPW // 2
_PL = 2 * _L
_LANES = 16
_NBUF = 4
_P = 1


def _sum_pool_sc(ids2, didx2, emb_table):
    mesh = plsc.VectorSubcoreMesh(core_axis_name="c", subcore_axis_name="s")

    @functools.partial(
        pl.kernel,
        out_type=jax.ShapeDtypeStruct((_B * _P, _H), jnp.float32),
        mesh=mesh,
        scratch_types=[
            pltpu.VMEM((_PAIRS, _PL), jnp.int32),
            pltpu.VMEM((_PAIRS, _PL), jnp.int32),
            pltpu.VMEM((_NBUF, _PL, _H), jnp.float32),
            pltpu.VMEM((_BPW, _H), jnp.float32),
            pltpu.VMEM_SHARED((_NS * _BPW * _P, _H), jnp.float32),
            [pltpu.SemaphoreType.DMA] * _NBUF,
            [pltpu.SemaphoreType.DMA] * _NBUF,
        ],
    )
    def body(ids_hbm, didx_hbm, table_hbm, out_hbm,
             ids_v, didx_v, rows_v, z_v, acc_sh, sems, ssems):
        c = lax.axis_index("c")
        s = lax.axis_index("s")
        wid = s * _NC + c
        pltpu.sync_copy(ids_hbm.at[pl.ds(wid * _PAIRS, _PAIRS)], ids_v)
        pltpu.sync_copy(didx_hbm.at[pl.ds(wid * _PAIRS, _PAIRS)], didx_v)

        def zbody(r, carry):
            for k in range(_H // _LANES):
                z_v[r, pl.ds(k * _LANES, _LANES)] = jnp.zeros(
                    (_LANES,), jnp.float32
                )
            return carry

        lax.fori_loop(0, _BPW, zbody, 0)
        for t in range(_P):
            pltpu.sync_copy(
                z_v, acc_sh.at[pl.ds(s * _BPW * _P + t * _BPW, _BPW)]
            )

        def issue(pair, buf):
            return pltpu.async_copy(
                table_hbm.at[ids_v.at[pair]], rows_v.at[buf], sems[buf]
            )

        def drain(pair, buf):
            pltpu.make_async_copy(
                table_hbm.at[ids_v.at[pair]], rows_v.at[buf], sems[buf]
            ).wait()

        def scat(pair, buf):
            return pltpu.async_copy(
                rows_v.at[buf], acc_sh.at[didx_v.at[pair]], ssems[buf], add=True
            )

        def scat_drain(pair, buf):
            pltpu.make_async_copy(
                rows_v.at[buf], acc_sh.at[didx_v.at[pair]], ssems[buf]
            ).wait()

        for b in range(_NBUF - 1):
            issue(b, b)

        def step(q, b, first):
            nb = (b + _NBUF - 1) % _NBUF
            if first:
                pl.when(q > 0)(lambda: scat_drain(q - 1, nb))
            else:
                scat_drain(q - 1, nb)
            issue(jnp.minimum(q + _NBUF - 1, _PAIRS - 1), nb)
            drain(q, b)
            scat(q, b)

        def loop_body(i, carry):
            for b in range(_NBUF):
                step(i * _NBUF + b, b, first=(b == 0))
            return carry

        lax.fori_loop(0, _PAIRS // _NBUF, loop_body, 0)
        scat_drain(_PAIRS - 1, (_PAIRS - 1) % _NBUF)
        for b in range(_NBUF - 1):
            drain(_PAIRS - 1, b)

        pltpu.sync_copy(
            acc_sh.at[pl.ds(s * _BPW * _P, _BPW * _P)],
            out_hbm.at[pl.ds(wid * _BPW * _P, _BPW * _P)],
        )

    return body(ids2, didx2, emb_table)


def _fc_body(emb_sum_ref, ids_ref, sym_ref, w1_ref, w2_ref, b_ref, out_ref):
    cnt = jnp.sum(jnp.where(ids_ref[...] != 0, 1.0, 0.0), axis=1, keepdims=True)
    avg = jnp.sum(emb_sum_ref[...], axis=1) * (1.0 / jnp.maximum(cnt, 1.0))
    out_ref[...] = (
        jnp.dot(avg, w1_ref[...], preferred_element_type=jnp.float32)
        + jnp.dot(sym_ref[...], w2_ref[...], preferred_element_type=jnp.float32)
        + b_ref[...]
    )


def kernel(ids, sym, emb_table, fc_w, fc_b):
    ids = ids.astype(jnp.int32)
    ids2 = ids.reshape(_B // 2, _PL)

    p = jnp.arange(_B // 2, dtype=jnp.int32)
    base = (p // _PAIRS // _NC) * _BPW + 2 * (p % _PAIRS)
    i = jnp.arange(_PL, dtype=jnp.int32)
    hi = (i >= _L).astype(jnp.int32)
    part = (i - _L * hi) % _P
    didx2 = (base[:, None] + hi[None, :]) * _P + part[None, :]

    emb_sum = _sum_pool_sc(ids2, didx2, emb_table).reshape(_B, _P, _H)

    w1 = fc_w[:, :_H].T
    w2 = fc_w[:, _H:].T
    out = pl.pallas_call(
        _fc_body,
        out_shape=jax.ShapeDtypeStruct((_B, _C), jnp.float32),
    )(emb_sum, ids, sym, w1, w2, fc_b.reshape(1, _C))
    return out


# speedup vs baseline: 1.0940x; 1.0196x over previous
import functools

import jax
import jax.numpy as jnp
import numpy as np
from jax import lax
from jax.experimental import pallas as pl
from jax.experimental.pallas import tpu as pltpu
from jax.experimental.pallas import tpu_sc as plsc

_B = 4096
_L = 50
_H = 128
_SYM = 32
_C = 16

_NC = 2
_NS = 16
_NW = _NC * _NS
_BPW = _B // _NW
_PAIRS = _BPW // 2
_PL = 2 * _L
_LANES = 16
_NBUF = 4
_P = 1


def _sum_pool_sc(ids2, didx2, emb_table):
    mesh = plsc.VectorSubcoreMesh(core_axis_name="c", subcore_axis_name="s")

    @functools.partial(
        pl.kernel,
        out_type=jax.ShapeDtypeStruct((_B * _P, _H), jnp.float32),
        mesh=mesh,
        scratch_types=[
            pltpu.VMEM((_PAIRS, _PL), jnp.int32),
            pltpu.VMEM((_PAIRS, _PL), jnp.int32),
            pltpu.VMEM((_NBUF, _PL, _H), jnp.float32),
            pltpu.VMEM((_BPW, _H), jnp.float32),
            pltpu.VMEM_SHARED((_NS * _BPW * _P, _H), jnp.float32),
            [pltpu.SemaphoreType.DMA] * _NBUF,
            [pltpu.SemaphoreType.DMA] * _NBUF,
        ],
    )
    def body(ids_hbm, didx_hbm, table_hbm, out_hbm,
             ids_v, didx_v, rows_v, z_v, acc_sh, sems, ssems):
        c = lax.axis_index("c")
        s = lax.axis_index("s")
        wid = s * _NC + c
        pltpu.sync_copy(ids_hbm.at[pl.ds(wid * _PAIRS, _PAIRS)], ids_v)
        pltpu.sync_copy(didx_hbm.at[pl.ds(wid * _PAIRS, _PAIRS)], didx_v)

        def zbody(r, carry):
            for k in range(_H // _LANES):
                z_v[r, pl.ds(k * _LANES, _LANES)] = jnp.zeros(
                    (_LANES,), jnp.float32
                )
            return carry

        lax.fori_loop(0, _BPW, zbody, 0)
        for t in range(_P):
            pltpu.sync_copy(
                z_v, acc_sh.at[pl.ds(s * _BPW * _P + t * _BPW, _BPW)]
            )

        def issue(pair, buf):
            return pltpu.async_copy(
                table_hbm.at[ids_v.at[pair]], rows_v.at[buf], sems[buf]
            )

        def drain(pair, buf):
            pltpu.make_async_copy(
                table_hbm.at[ids_v.at[pair]], rows_v.at[buf], sems[buf]
            ).wait()

        def scat(pair, buf):
            return pltpu.async_copy(
                rows_v.at[buf], acc_sh.at[didx_v.at[pair]], ssems[buf], add=True
            )

        def scat_drain(pair, buf):
            pltpu.make_async_copy(
                rows_v.at[buf], acc_sh.at[didx_v.at[pair]], ssems[buf]
            ).wait()

        for b in range(_NBUF - 1):
            issue(b, b)

        def step(q, b, first):
            nb = (b + _NBUF - 1) % _NBUF
            if first:
                pl.when(q > 0)(lambda: scat_drain(q - 1, nb))
            else:
                scat_drain(q - 1, nb)
            issue(jnp.minimum(q + _NBUF - 1, _PAIRS - 1), nb)
            drain(q, b)
            scat(q, b)

        def loop_body(i, carry):
            for b in range(_NBUF):
                step(i * _NBUF + b, b, first=(b == 0))
            return carry

        lax.fori_loop(0, _PAIRS // _NBUF, loop_body, 0)
        scat_drain(_PAIRS - 1, (_PAIRS - 1) % _NBUF)
        for b in range(_NBUF - 1):
            drain(_PAIRS - 1, b)

        pltpu.sync_copy(
            acc_sh.at[pl.ds(s * _BPW * _P, _BPW * _P)],
            out_hbm.at[pl.ds(wid * _BPW * _P, _BPW * _P)],
        )

    return body(ids2, didx2, emb_table)


def _fc_body(emb_sum_ref, ids_ref, sym_ref, w_ref, b_ref, out_ref):
    cnt = jnp.sum(jnp.where(ids_ref[...] != 0, 1.0, 0.0), axis=1, keepdims=True)
    avg = emb_sum_ref[...] * (1.0 / jnp.maximum(cnt, 1.0))
    dims = (((1,), (1,)), ((), ()))
    out_ref[...] = (
        lax.dot_general(avg, w_ref[:, :_H], dims,
                        preferred_element_type=jnp.float32)
        + lax.dot_general(sym_ref[...], w_ref[:, _H:], dims,
                          preferred_element_type=jnp.float32)
        + b_ref[...]
    )


_p = np.arange(_B // 2, dtype=np.int32)
_base = (_p // _PAIRS // _NC) * _BPW + 2 * (_p % _PAIRS)
_i = np.arange(_PL, dtype=np.int32)
_hi = (_i >= _L).astype(np.int32)
_part = (_i - _L * _hi) % _P
_DIDX2 = jnp.asarray((_base[:, None] + _hi[None, :]) * _P + _part[None, :])


def kernel(ids, sym, emb_table, fc_w, fc_b):
    ids = ids.astype(jnp.int32)
    ids2 = ids.reshape(_B // 2, _PL)

    emb_sum = _sum_pool_sc(ids2, _DIDX2, emb_table)

    out = pl.pallas_call(
        _fc_body,
        out_shape=jax.ShapeDtypeStruct((_B, _C), jnp.float32),
    )(emb_sum, ids, sym, fc_w, fc_b.reshape(1, _C))
    return out


# speedup vs baseline: 1.1074x; 1.0122x over previous
import functools

import jax
import jax.numpy as jnp
import numpy as np
from jax import lax
from jax.experimental import pallas as pl
from jax.experimental.pallas import tpu as pltpu
from jax.experimental.pallas import tpu_sc as plsc

_B = 4096
_L = 50
_H = 128
_SYM = 32
_C = 16

_NC = 2
_NS = 16
_NW = _NC * _NS
_BPW = _B // _NW
_PAIRS = _BPW // 2
_PL = 2 * _L
_LANES = 16
_NBUF = 4
_P = 1


def _sum_pool_sc(ids2, didx2, emb_table):
    mesh = plsc.VectorSubcoreMesh(core_axis_name="c", subcore_axis_name="s")

    @functools.partial(
        pl.kernel,
        out_type=jax.ShapeDtypeStruct((_B * _P, _H), jnp.float32),
        mesh=mesh,
        scratch_types=[
            pltpu.VMEM((_PAIRS, _PL), jnp.int32),
            pltpu.VMEM((_PAIRS, _PL), jnp.int32),
            pltpu.VMEM((_NBUF, _PL, _H), jnp.float32),
            pltpu.VMEM((_BPW, _H), jnp.float32),
            pltpu.VMEM_SHARED((_NS * _BPW * _P, _H), jnp.float32),
            [pltpu.SemaphoreType.DMA] * _NBUF,
            [pltpu.SemaphoreType.DMA] * _NBUF,
        ],
    )
    def body(ids_hbm, didx_hbm, table_hbm, out_hbm,
             ids_v, didx_v, rows_v, z_v, acc_sh, sems, ssems):
        c = lax.axis_index("c")
        s = lax.axis_index("s")
        wid = s * _NC + c
        pltpu.sync_copy(ids_hbm.at[pl.ds(wid * _PAIRS, _PAIRS)], ids_v)
        pltpu.sync_copy(didx_hbm.at[pl.ds(wid * _PAIRS, _PAIRS)], didx_v)

        def zbody(r, carry):
            for k in range(_H // _LANES):
                z_v[r, pl.ds(k * _LANES, _LANES)] = jnp.zeros(
                    (_LANES,), jnp.float32
                )
            return carry

        lax.fori_loop(0, _BPW, zbody, 0)
        for t in range(_P):
            pltpu.sync_copy(
                z_v, acc_sh.at[pl.ds(s * _BPW * _P + t * _BPW, _BPW)]
            )

        def issue(pair, buf):
            return pltpu.async_copy(
                table_hbm.at[ids_v.at[pair]], rows_v.at[buf], sems[buf]
            )

        def drain(pair, buf):
            pltpu.make_async_copy(
                table_hbm.at[ids_v.at[pair]], rows_v.at[buf], sems[buf]
            ).wait()

        def scat(pair, buf):
            return pltpu.async_copy(
                rows_v.at[buf], acc_sh.at[didx_v.at[pair]], ssems[buf], add=True
            )

        def scat_drain(pair, buf):
            pltpu.make_async_copy(
                rows_v.at[buf], acc_sh.at[didx_v.at[pair]], ssems[buf]
            ).wait()

        for b in range(_NBUF - 1):
            issue(b, b)

        def step(q, b, first):
            nb = (b + _NBUF - 1) % _NBUF
            if first:
                pl.when(q > 0)(lambda: scat_drain(q - 1, nb))
            else:
                scat_drain(q - 1, nb)
            qq = q + _NBUF - 1

            def _issue_tail():
                issue(qq, nb)

            pl.when(qq < _PAIRS)(_issue_tail)
            drain(q, b)
            scat(q, b)

        def loop_body(i, carry):
            for b in range(_NBUF):
                step(i * _NBUF + b, b, first=(b == 0))
            return carry

        lax.fori_loop(0, _PAIRS // _NBUF, loop_body, 0)
        scat_drain(_PAIRS - 1, (_PAIRS - 1) % _NBUF)

        pltpu.sync_copy(
            acc_sh.at[pl.ds(s * _BPW * _P, _BPW * _P)],
            out_hbm.at[pl.ds(wid * _BPW * _P, _BPW * _P)],
        )

    return body(ids2, didx2, emb_table)


def _fc_body(emb_sum_ref, ids_ref, sym_ref, w_ref, b_ref, out_ref):
    cnt = jnp.sum(jnp.where(ids_ref[...] != 0, 1.0, 0.0), axis=1, keepdims=True)
    avg = emb_sum_ref[...] * (1.0 / jnp.maximum(cnt, 1.0))
    dims = (((1,), (1,)), ((), ()))
    out_ref[...] = (
        lax.dot_general(avg, w_ref[:, :_H], dims,
                        preferred_element_type=jnp.float32)
        + lax.dot_general(sym_ref[...], w_ref[:, _H:], dims,
                          preferred_element_type=jnp.float32)
        + b_ref[...]
    )


_p = np.arange(_B // 2, dtype=np.int32)
_base = (_p // _PAIRS // _NC) * _BPW + 2 * (_p % _PAIRS)
_i = np.arange(_PL, dtype=np.int32)
_hi = (_i >= _L).astype(np.int32)
_part = (_i - _L * _hi) % _P
_DIDX2 = jnp.asarray((_base[:, None] + _hi[None, :]) * _P + _part[None, :])


def kernel(ids, sym, emb_table, fc_w, fc_b):
    ids = ids.astype(jnp.int32)
    ids2 = ids.reshape(_B // 2, _PL)

    emb_sum = _sum_pool_sc(ids2, _DIDX2, emb_table)

    out = pl.pallas_call(
        _fc_body,
        out_shape=jax.ShapeDtypeStruct((_B, _C), jnp.float32),
    )(emb_sum, ids, sym, fc_w, fc_b.reshape(1, _C))
    return out
